# baseline (speedup 1.0000x reference)
import functools
import math

import jax
import jax.numpy as jnp
from jax import lax
from jax.experimental import pallas as pl
from jax.experimental.pallas import tpu as pltpu
from jax.experimental.pallas import tpu_sc as plsc

_LANES = 16
_IDX_CHUNK = 128


def _pool_sc(idx3, embedding, batch, ctx):
    d = embedding.shape[1]
    info = plsc.get_sparse_core_info()
    nw = info.num_cores * info.num_subcores
    n_chunks = idx3.shape[1]
    b_per_w = batch // nw
    g_per_w = n_chunks * _IDX_CHUNK

    mesh = plsc.VectorSubcoreMesh(core_axis_name="c", subcore_axis_name="s")

    @functools.partial(
        pl.kernel,
        mesh=mesh,
        out_type=jax.ShapeDtypeStruct((batch, d), jnp.float32),
        scratch_types=[
            pltpu.VMEM((n_chunks, _IDX_CHUNK), jnp.int32),
            pltpu.VMEM((g_per_w, d), jnp.float32),
            pltpu.VMEM((b_per_w, d), jnp.float32),
            pltpu.SemaphoreType.DMA,
        ],
        compiler_params=pltpu.CompilerParams(use_tc_tiling_on_sc=False),
    )
    def pool(idx_hbm, emb_hbm, out_hbm, idx_v, rows_v, acc_v, sem):
        wid = lax.axis_index("s") * info.num_cores + lax.axis_index("c")
        pltpu.sync_copy(idx_hbm.at[wid], idx_v)
        copies = [
            pltpu.async_copy(
                emb_hbm.at[idx_v.at[j]],
                rows_v.at[pl.ds(j * _IDX_CHUNK, _IDX_CHUNK)],
                sem,
            )
            for j in range(n_chunks)
        ]
        for cp in copies:
            cp.wait()

        def body(b, carry):
            g0 = b * ctx
            for dd in range(d // _LANES):
                sl = pl.ds(dd * _LANES, _LANES)
                s = rows_v[g0, sl]
                for c in range(1, ctx):
                    s = s + rows_v[g0 + c, sl]
                acc_v[b, sl] = s
            return carry

        lax.fori_loop(0, b_per_w, body, 0)
        pltpu.sync_copy(acc_v, out_hbm.at[pl.ds(wid * b_per_w, b_per_w)])

    return pool(idx3, embedding)


_NBUF = 5


def _make_mm_body(v, vb, nb, nbuf):
    tail = ((v - (nb - 1) * vb) // 128) * 128

    def _mm_body(x_ref, w_ref, b_ref, o_hbm, *scratch):
        bufs, sems = scratch[:nbuf], scratch[nbuf:]
        j = pl.program_id(0)
        y = (
            lax.dot_general(
                x_ref[...], w_ref[...],
                (((1,), (1,)), ((), ())),
                preferred_element_type=jnp.float32,
            )
            + b_ref[0]
        )
        for k in range(nbuf):
            @pl.when(j % nbuf == k)
            def _():
                @pl.when(j >= nbuf)
                def _():
                    pltpu.make_async_copy(
                        bufs[k], o_hbm.at[:, pl.ds(j * vb, vb)], sems[k]
                    ).wait()

                bufs[k][...] = y

                @pl.when(j < nb - 1)
                def _():
                    pltpu.make_async_copy(
                        bufs[k], o_hbm.at[:, pl.ds(j * vb, vb)], sems[k]
                    ).start()

                @pl.when(j == nb - 1)
                def _():
                    pltpu.make_async_copy(
                        bufs[k].at[:, pl.ds(0, tail)],
                        o_hbm.at[:, pl.ds(j * vb, tail)],
                        sems[k],
                    ).start()

        @pl.when(j == pl.num_programs(0) - 1)
        def _():
            for k in range(nbuf):
                jj = nb - 1 - (nb - 1 - k) % nbuf
                w_k = tail if jj == nb - 1 else vb
                pltpu.make_async_copy(
                    bufs[k].at[:, pl.ds(0, w_k)],
                    o_hbm.at[:, pl.ds(jj * vb, w_k)],
                    sems[k],
                ).wait()

    return _mm_body


def _fc1_tc(x_sum, fc1_w, fc1_b, vb=2048):
    batch, d = x_sum.shape
    v = fc1_w.shape[0]
    nb = math.ceil(v / vb)
    b_pad = jnp.zeros((nb * vb,), jnp.float32).at[:v].set(fc1_b)
    return pl.pallas_call(
        _make_mm_body(v, vb, nb, _NBUF),
        grid=(nb,),
        in_specs=[
            pl.BlockSpec((batch, d), lambda j: (0, 0)),
            pl.BlockSpec((vb, d), lambda j: (j, 0)),
            pl.BlockSpec((1, 1, vb), lambda j: (j, 0, 0)),
        ],
        out_specs=pl.BlockSpec(memory_space=pl.ANY),
        out_shape=jax.ShapeDtypeStruct((batch, v), jnp.float32),
        scratch_shapes=(
            [pltpu.VMEM((batch, vb), jnp.float32) for _ in range(_NBUF)]
            + [pltpu.SemaphoreType.DMA for _ in range(_NBUF)]
        ),
        compiler_params=pltpu.CompilerParams(
            dimension_semantics=("arbitrary",),
        ),
    )(x_sum, fc1_w, b_pad.reshape(nb, 1, vb))


def kernel(x_in, embedding, fc1_w, fc1_b):
    batch, ctx = x_in.shape
    info = plsc.get_sparse_core_info()
    nw = info.num_cores * info.num_subcores
    g_per_w = (batch // nw) * ctx
    n_chunks = g_per_w // _IDX_CHUNK
    idx3 = x_in.astype(jnp.int32).reshape(nw, n_chunks, _IDX_CHUNK)
    x_sum = _pool_sc(idx3, embedding, batch, ctx)
    return _fc1_tc(x_sum, fc1_w, fc1_b)


# speedup vs baseline: 2.1423x; 2.1423x over previous
import functools
import math

import jax
import jax.numpy as jnp
from jax import lax
from jax.experimental import pallas as pl
from jax.experimental.pallas import tpu as pltpu
from jax.experimental.pallas import tpu_sc as plsc

_LANES = 16
_IDX_CHUNK = 128


def _pool_sc(idx3, embedding, batch, ctx):
    d = embedding.shape[1]
    info = plsc.get_sparse_core_info()
    nw = info.num_cores * info.num_subcores
    n_chunks = idx3.shape[1]
    b_per_w = batch // nw
    g_per_w = n_chunks * _IDX_CHUNK

    mesh = plsc.VectorSubcoreMesh(core_axis_name="c", subcore_axis_name="s")

    @functools.partial(
        pl.kernel,
        mesh=mesh,
        out_type=jax.ShapeDtypeStruct((batch, d), jnp.float32),
        scratch_types=[
            pltpu.VMEM((n_chunks, _IDX_CHUNK), jnp.int32),
            pltpu.VMEM((g_per_w, d), jnp.float32),
            pltpu.VMEM((b_per_w, d), jnp.float32),
            pltpu.SemaphoreType.DMA,
        ],
        compiler_params=pltpu.CompilerParams(use_tc_tiling_on_sc=False),
    )
    def pool(idx_hbm, emb_hbm, out_hbm, idx_v, rows_v, acc_v, sem):
        wid = lax.axis_index("s") * info.num_cores + lax.axis_index("c")
        pltpu.sync_copy(idx_hbm.at[wid], idx_v)
        copies = [
            pltpu.async_copy(
                emb_hbm.at[idx_v.at[j]],
                rows_v.at[pl.ds(j * _IDX_CHUNK, _IDX_CHUNK)],
                sem,
            )
            for j in range(n_chunks)
        ]
        for cp in copies:
            cp.wait()

        def body(b, carry):
            g0 = b * ctx
            for dd in range(d // _LANES):
                sl = pl.ds(dd * _LANES, _LANES)
                s = rows_v[g0, sl]
                for c in range(1, ctx):
                    s = s + rows_v[g0 + c, sl]
                acc_v[b, sl] = s
            return carry

        lax.fori_loop(0, b_per_w, body, 0)
        pltpu.sync_copy(acc_v, out_hbm.at[pl.ds(wid * b_per_w, b_per_w)])

    return pool(idx3, embedding)


def _mm_body(x_ref, wt_ref, b_ref, o_ref):
    o_ref[...] = (
        lax.dot_general(
            wt_ref[...], x_ref[...],
            (((0,), (1,)), ((), ())),
            preferred_element_type=jnp.float32,
        )
        + b_ref[0]
    )


def _fc1_tc(x_sum, fc1_w, fc1_b, vb=2048):
    batch, d = x_sum.shape
    v = fc1_w.shape[0]
    nb = math.ceil(v / vb)
    wt = fc1_w.T
    b_pad = jnp.zeros((nb * vb,), jnp.float32).at[:v].set(fc1_b)
    return pl.pallas_call(
        _mm_body,
        grid=(nb,),
        in_specs=[
            pl.BlockSpec((batch, d), lambda j: (0, 0)),
            pl.BlockSpec((d, vb), lambda j: (0, j)),
            pl.BlockSpec((1, vb, 1), lambda j: (j, 0, 0)),
        ],
        out_specs=pl.BlockSpec((vb, batch), lambda j: (j, 0)),
        out_shape=jax.ShapeDtypeStruct((v, batch), jnp.float32),
        compiler_params=pltpu.CompilerParams(
            dimension_semantics=("arbitrary",),
        ),
    )(x_sum, wt, b_pad.reshape(nb, vb, 1))


def kernel(x_in, embedding, fc1_w, fc1_b):
    batch, ctx = x_in.shape
    info = plsc.get_sparse_core_info()
    nw = info.num_cores * info.num_subcores
    g_per_w = (batch // nw) * ctx
    n_chunks = g_per_w // _IDX_CHUNK
    idx3 = x_in.astype(jnp.int32).reshape(nw, n_chunks, _IDX_CHUNK)
    x_sum = _pool_sc(idx3, embedding, batch, ctx)
    return _fc1_tc(x_sum, fc1_w, fc1_b).T


# speedup vs baseline: 2.7529x; 1.2850x over previous
import functools
import math

import jax
import jax.numpy as jnp
from jax import lax
from jax.experimental import pallas as pl
from jax.experimental.pallas import tpu as pltpu
from jax.experimental.pallas import tpu_sc as plsc

_LANES = 16
_IDX_CHUNK = 128


def _pool_sc(idx3, embedding, batch, ctx):
    d = embedding.shape[1]
    info = plsc.get_sparse_core_info()
    nw = info.num_cores * info.num_subcores
    n_chunks = idx3.shape[1]
    b_per_w = batch // nw
    g_per_w = n_chunks * _IDX_CHUNK

    mesh = plsc.VectorSubcoreMesh(core_axis_name="c", subcore_axis_name="s")

    @functools.partial(
        pl.kernel,
        mesh=mesh,
        out_type=jax.ShapeDtypeStruct((batch, d), jnp.float32),
        scratch_types=[
            pltpu.VMEM((n_chunks, _IDX_CHUNK), jnp.int32),
            pltpu.VMEM((g_per_w, d), jnp.float32),
            pltpu.VMEM((b_per_w, d), jnp.float32),
            pltpu.SemaphoreType.DMA,
        ],
        compiler_params=pltpu.CompilerParams(use_tc_tiling_on_sc=False),
    )
    def pool(idx_hbm, emb_hbm, out_hbm, idx_v, rows_v, acc_v, sem):
        wid = lax.axis_index("s") * info.num_cores + lax.axis_index("c")
        pltpu.sync_copy(idx_hbm.at[wid], idx_v)
        copies = [
            pltpu.async_copy(
                emb_hbm.at[idx_v.at[j]],
                rows_v.at[pl.ds(j * _IDX_CHUNK, _IDX_CHUNK)],
                sem,
            )
            for j in range(n_chunks)
        ]
        for cp in copies:
            cp.wait()

        def body(b, carry):
            g0 = b * ctx
            for dd in range(d // _LANES):
                sl = pl.ds(dd * _LANES, _LANES)
                s = rows_v[g0, sl]
                for c in range(1, ctx):
                    s = s + rows_v[g0 + c, sl]
                acc_v[b, sl] = s
            return carry

        lax.fori_loop(0, b_per_w, body, 0)
        pltpu.sync_copy(acc_v, out_hbm.at[pl.ds(wid * b_per_w, b_per_w)])

    return pool(idx3, embedding)


def _mm_body(x_ref, wt_ref, b_ref, o_ref):
    o_ref[...] = (
        lax.dot_general(
            wt_ref[...], x_ref[...],
            (((0,), (1,)), ((), ())),
            preferred_element_type=jnp.float32,
        )
        + b_ref[0, 0][:, None]
    )


def _fc1_tc(x_sum, fc1_w, fc1_b, vb=2048):
    batch, d = x_sum.shape
    v = fc1_w.shape[0]
    nb = math.ceil(v / vb)
    wt = fc1_w.T
    b_pad = jnp.zeros((nb * vb,), jnp.float32).at[:v].set(fc1_b)
    return pl.pallas_call(
        _mm_body,
        grid=(nb,),
        in_specs=[
            pl.BlockSpec((batch, d), lambda j: (0, 0)),
            pl.BlockSpec((d, vb), lambda j: (0, j)),
            pl.BlockSpec((1, 1, vb), lambda j: (j, 0, 0)),
        ],
        out_specs=pl.BlockSpec((vb, batch), lambda j: (j, 0)),
        out_shape=jax.ShapeDtypeStruct((v, batch), jnp.float32),
        compiler_params=pltpu.CompilerParams(
            dimension_semantics=("arbitrary",),
        ),
    )(x_sum, wt, b_pad.reshape(nb, 1, vb))


def kernel(x_in, embedding, fc1_w, fc1_b):
    batch, ctx = x_in.shape
    info = plsc.get_sparse_core_info()
    nw = info.num_cores * info.num_subcores
    g_per_w = (batch // nw) * ctx
    n_chunks = g_per_w // _IDX_CHUNK
    idx3 = x_in.astype(jnp.int32).reshape(nw, n_chunks, _IDX_CHUNK)
    x_sum = _pool_sc(idx3, embedding, batch, ctx)
    return _fc1_tc(x_sum, fc1_w, fc1_b).T


# speedup vs baseline: 2.9920x; 1.0869x over previous
import functools
import math

import jax
import jax.numpy as jnp
from jax import lax
from jax.experimental import pallas as pl
from jax.experimental.pallas import tpu as pltpu
from jax.experimental.pallas import tpu_sc as plsc

_LANES = 16


def _pool_sc(idx_flat, emb_flat, batch, ctx, d, v):
    info = plsc.get_sparse_core_info()
    nw = info.num_cores * info.num_subcores
    rows_per_w = d // nw if d >= nw else 1
    n_chunks = batch // _LANES

    mesh = plsc.VectorSubcoreMesh(core_axis_name="c", subcore_axis_name="s")

    @functools.partial(
        pl.kernel,
        mesh=mesh,
        out_type=jax.ShapeDtypeStruct((d * batch,), jnp.float32),
        scratch_types=[
            pltpu.VMEM((ctx * batch,), jnp.int32),
            pltpu.VMEM((v,), jnp.float32),
            pltpu.VMEM((rows_per_w * batch,), jnp.float32),
            pltpu.SemaphoreType.DMA,
        ],
        compiler_params=pltpu.CompilerParams(
            use_tc_tiling_on_sc=False, needs_layout_passes=False
        ),
    )
    def pool(idx_hbm, emb_hbm, out_hbm, idx_v, row_v, acc_v, sem):
        wid = lax.axis_index("s") * info.num_cores + lax.axis_index("c")
        pltpu.sync_copy(idx_hbm, idx_v)
        for i in range(rows_per_w):
            r = wid * rows_per_w + i
            pltpu.async_copy(emb_hbm.at[pl.ds(r * v, v)], row_v, sem).wait()

            def body(k, carry):
                acc = plsc.load_gather(row_v, [idx_v[pl.ds(k * _LANES, _LANES)]])
                for c in range(1, ctx):
                    acc = acc + plsc.load_gather(
                        row_v, [idx_v[pl.ds(c * batch + k * _LANES, _LANES)]]
                    )
                acc_v[pl.ds(i * batch + k * _LANES, _LANES)] = acc
                return carry

            lax.fori_loop(0, n_chunks, body, 0)
        pltpu.sync_copy(
            acc_v, out_hbm.at[pl.ds(wid * rows_per_w * batch, rows_per_w * batch)]
        )

    return pool(idx_flat, emb_flat)


def _mm_body(xt_ref, wt_ref, b_ref, o_ref):
    o_ref[...] = (
        lax.dot_general(
            wt_ref[...], xt_ref[...],
            (((0,), (0,)), ((), ())),
            preferred_element_type=jnp.float32,
        )
        + b_ref[0, 0][:, None]
    )


def _fc1_tc(x_sum_t, fc1_w, fc1_b, vb=2048):
    d, batch = x_sum_t.shape
    v = fc1_w.shape[0]
    nb = math.ceil(v / vb)
    wt = fc1_w.T
    b_pad = jnp.zeros((nb * vb,), jnp.float32).at[:v].set(fc1_b)
    return pl.pallas_call(
        _mm_body,
        grid=(nb,),
        in_specs=[
            pl.BlockSpec((d, batch), lambda j: (0, 0)),
            pl.BlockSpec((d, vb), lambda j: (0, j)),
            pl.BlockSpec((1, 1, vb), lambda j: (j, 0, 0)),
        ],
        out_specs=pl.BlockSpec((vb, batch), lambda j: (j, 0)),
        out_shape=jax.ShapeDtypeStruct((v, batch), jnp.float32),
        compiler_params=pltpu.CompilerParams(
            dimension_semantics=("arbitrary",),
        ),
    )(x_sum_t, wt, b_pad.reshape(nb, 1, vb))


def kernel(x_in, embedding, fc1_w, fc1_b):
    batch, ctx = x_in.shape
    v, d = embedding.shape
    idx_flat = x_in.astype(jnp.int32).T.reshape(-1)
    emb_flat = embedding.T.reshape(-1)
    x_sum_t = _pool_sc(idx_flat, emb_flat, batch, ctx, d, v).reshape(d, batch)
    return _fc1_tc(x_sum_t, fc1_w, fc1_b).T
